# Initial kernel scaffold; baseline (speedup 1.0000x reference)
import jax
import jax.numpy as jnp
from jax.experimental import pallas as pl


def kernel(features, labels, centers):
    raise NotImplementedError("write your pallas kernel here")


# speedup vs baseline: 1.0793x; 1.0793x over previous
import functools

import jax
import jax.numpy as jnp
from jax import lax
from jax.experimental import pallas as pl
from jax.experimental.pallas import tpu as pltpu
from jax.experimental.pallas import tpu_sc as plsc

_NUM_CLASSES = 100000
_FEAT_DIM = 128
_BATCH = 16384
_LAMBDA_C = 0.001

_NC = 2
_NS = 16
_NW = _NC * _NS
_PER_W = _BATCH // _NW
_CHUNK = 128
_NCHUNK = _PER_W // _CHUNK
_L = 16


def _sc_body(feats_hbm, labels_hbm, centers_hbm, out_hbm,
             idx_v, feat_v, rows_v, acc_v, gsem, fsem):
    wid = lax.axis_index("s") * _NC + lax.axis_index("c")
    base = wid * _PER_W

    pltpu.sync_copy(labels_hbm.at[pl.ds(base, _PER_W)], idx_v)

    acc = jnp.zeros((_L,), jnp.float32)
    for c in range(_NCHUNK):
        row0 = base + c * _CHUNK
        gcp = pltpu.async_copy(
            centers_hbm.at[idx_v.at[pl.ds(c * _CHUNK, _CHUNK)]], rows_v, gsem)
        fcp = pltpu.async_copy(feats_hbm.at[pl.ds(row0, _CHUNK)], feat_v, fsem)
        gcp.wait()
        fcp.wait()

        def row_body(i, a):
            for j in range(_FEAT_DIM // _L):
                d = feat_v[i, pl.ds(j * _L, _L)] - rows_v[i, pl.ds(j * _L, _L)]
                a = a + d * d
            return a

        acc = lax.fori_loop(0, _CHUNK, row_body, acc)

    acc_v[...] = acc
    pltpu.sync_copy(acc_v, out_hbm.at[wid])


@functools.partial(jax.jit, static_argnames=())
def _center_loss_sc(features, labels_i32, centers):
    mesh = plsc.VectorSubcoreMesh(core_axis_name="c", subcore_axis_name="s")
    partials = pl.kernel(
        _sc_body,
        out_type=jax.ShapeDtypeStruct((_NW, _L), jnp.float32),
        mesh=mesh,
        scratch_types=[
            pltpu.VMEM((_PER_W,), jnp.int32),
            pltpu.VMEM((_CHUNK, _FEAT_DIM), jnp.float32),
            pltpu.VMEM((_CHUNK, _FEAT_DIM), jnp.float32),
            pltpu.VMEM((_L,), jnp.float32),
            pltpu.SemaphoreType.DMA,
            pltpu.SemaphoreType.DMA,
        ],
    )(features, labels_i32, centers)
    scale = _LAMBDA_C / float(_BATCH * _FEAT_DIM)
    return jnp.sum(partials) * scale


def kernel(features, labels, centers):
    return _center_loss_sc(features, labels.astype(jnp.int32), centers)


# speedup vs baseline: 1.2000x; 1.1118x over previous
import functools

import jax
import jax.numpy as jnp
from jax import lax
from jax.experimental import pallas as pl
from jax.experimental.pallas import tpu as pltpu
from jax.experimental.pallas import tpu_sc as plsc

_NUM_CLASSES = 100000
_FEAT_DIM = 128
_BATCH = 16384
_LAMBDA_C = 0.001

_NC = 2
_NS = 16
_NW = _NC * _NS
_PER_W = _BATCH // _NW
_CHUNK = 128
_NCHUNK = _PER_W // _CHUNK
_L = 16


_NJ = _FEAT_DIM // _L


def _sc_body(feats_hbm, labels_hbm, centers_hbm, out_hbm,
             idx_v, feat0, feat1, rows0, rows1, acc_v,
             gsem0, gsem1, fsem0, fsem1):
    wid = lax.axis_index("s") * _NC + lax.axis_index("c")
    base = wid * _PER_W

    pltpu.sync_copy(labels_hbm.at[pl.ds(base, _PER_W)], idx_v)

    feats = (feat0, feat1)
    rows = (rows0, rows1)
    gsems = (gsem0, gsem1)
    fsems = (fsem0, fsem1)

    def start(c):
        b = c % 2
        gcp = pltpu.async_copy(
            centers_hbm.at[idx_v.at[pl.ds(c * _CHUNK, _CHUNK)]],
            rows[b], gsems[b])
        fcp = pltpu.async_copy(
            feats_hbm.at[pl.ds(base + c * _CHUNK, _CHUNK)],
            feats[b], fsems[b])
        return gcp, fcp

    copies = {0: start(0)}
    accs = tuple(jnp.zeros((_L,), jnp.float32) for _ in range(_NJ))
    for c in range(_NCHUNK):
        if c + 1 < _NCHUNK:
            copies[c + 1] = start(c + 1)
        gcp, fcp = copies.pop(c)
        gcp.wait()
        fcp.wait()
        b = c % 2
        f_v, r_v = feats[b], rows[b]

        @plsc.parallel_loop(0, _CHUNK, carry=accs)
        def accs(i, a):
            out = []
            for j in range(_NJ):
                d = f_v[i, pl.ds(j * _L, _L)] - r_v[i, pl.ds(j * _L, _L)]
                out.append(a[j] + d * d)
            return tuple(out)

    total = accs[0]
    for j in range(1, _NJ):
        total = total + accs[j]
    acc_v[...] = total
    pltpu.sync_copy(acc_v, out_hbm.at[wid])


@functools.partial(jax.jit, static_argnames=())
def _center_loss_sc(features, labels_i32, centers):
    mesh = plsc.VectorSubcoreMesh(core_axis_name="c", subcore_axis_name="s")
    partials = pl.kernel(
        _sc_body,
        out_type=jax.ShapeDtypeStruct((_NW, _L), jnp.float32),
        mesh=mesh,
        scratch_types=[
            pltpu.VMEM((_PER_W,), jnp.int32),
            pltpu.VMEM((_CHUNK, _FEAT_DIM), jnp.float32),
            pltpu.VMEM((_CHUNK, _FEAT_DIM), jnp.float32),
            pltpu.VMEM((_CHUNK, _FEAT_DIM), jnp.float32),
            pltpu.VMEM((_CHUNK, _FEAT_DIM), jnp.float32),
            pltpu.VMEM((_L,), jnp.float32),
            pltpu.SemaphoreType.DMA,
            pltpu.SemaphoreType.DMA,
            pltpu.SemaphoreType.DMA,
            pltpu.SemaphoreType.DMA,
        ],
    )(features, labels_i32, centers)
    scale = _LAMBDA_C / float(_BATCH * _FEAT_DIM)
    return jnp.sum(partials) * scale


def kernel(features, labels, centers):
    return _center_loss_sc(features, labels.astype(jnp.int32), centers)


# speedup vs baseline: 1.2854x; 1.0712x over previous
import functools

import jax
import jax.numpy as jnp
from jax import lax
from jax.experimental import pallas as pl
from jax.experimental.pallas import tpu as pltpu
from jax.experimental.pallas import tpu_sc as plsc

_NUM_CLASSES = 100000
_FEAT_DIM = 128
_BATCH = 16384
_LAMBDA_C = 0.001

_NC = 2
_NS = 16
_NW = _NC * _NS
_PER_W = _BATCH // _NW
_CHUNK = 64
_NCHUNK = _PER_W // _CHUNK
_NBUF = 4
_L = 16


_NJ = _FEAT_DIM // _L


def _sc_body(feats_hbm, labels_hbm, centers_hbm, out_hbm,
             idx_v, acc_v, *bufs_and_sems):
    feats = bufs_and_sems[0:_NBUF]
    rows = bufs_and_sems[_NBUF:2 * _NBUF]
    gsems = bufs_and_sems[2 * _NBUF:3 * _NBUF]
    fsems = bufs_and_sems[3 * _NBUF:4 * _NBUF]

    wid = lax.axis_index("s") * _NC + lax.axis_index("c")
    base = wid * _PER_W

    pltpu.sync_copy(labels_hbm.at[pl.ds(base, _PER_W)], idx_v)

    def start(c):
        b = c % _NBUF
        gcp = pltpu.async_copy(
            centers_hbm.at[idx_v.at[pl.ds(c * _CHUNK, _CHUNK)]],
            rows[b], gsems[b])
        fcp = pltpu.async_copy(
            feats_hbm.at[pl.ds(base + c * _CHUNK, _CHUNK)],
            feats[b], fsems[b])
        return gcp, fcp

    copies = {c: start(c) for c in range(min(_NBUF, _NCHUNK))}
    accs = tuple(jnp.zeros((_L,), jnp.float32) for _ in range(_NJ))
    for c in range(_NCHUNK):
        gcp, fcp = copies.pop(c)
        gcp.wait()
        fcp.wait()
        b = c % _NBUF
        f_v, r_v = feats[b], rows[b]

        @plsc.parallel_loop(0, _CHUNK, carry=accs)
        def accs(i, a):
            out = []
            for j in range(_NJ):
                d = f_v[i, pl.ds(j * _L, _L)] - r_v[i, pl.ds(j * _L, _L)]
                out.append(a[j] + d * d)
            return tuple(out)

        if c + _NBUF < _NCHUNK:
            copies[c + _NBUF] = start(c + _NBUF)

    total = accs[0]
    for j in range(1, _NJ):
        total = total + accs[j]
    acc_v[...] = total * (_LAMBDA_C / float(_BATCH * _FEAT_DIM))
    pltpu.sync_copy(acc_v, out_hbm.at[wid])


@functools.partial(jax.jit, static_argnames=())
def _center_loss_sc(features, labels_i32, centers):
    mesh = plsc.VectorSubcoreMesh(core_axis_name="c", subcore_axis_name="s")
    partials = pl.kernel(
        _sc_body,
        out_type=jax.ShapeDtypeStruct((_NW, _L), jnp.float32),
        mesh=mesh,
        scratch_types=(
            [pltpu.VMEM((_PER_W,), jnp.int32),
             pltpu.VMEM((_L,), jnp.float32)]
            + [pltpu.VMEM((_CHUNK, _FEAT_DIM), jnp.float32)
               for _ in range(2 * _NBUF)]
            + [pltpu.SemaphoreType.DMA for _ in range(2 * _NBUF)]
        ),
    )(features, labels_i32, centers)
    return jnp.sum(partials)


def kernel(features, labels, centers):
    return _center_loss_sc(features, labels.astype(jnp.int32), centers)
